# baseline (speedup 1.0000x reference)
import functools

import jax
import jax.numpy as jnp
from jax import lax
from jax.experimental import pallas as pl
from jax.experimental.pallas import tpu as pltpu
from jax.experimental.pallas import tpu_sc as plsc

NC = 2
NS = 16
NW = NC * NS
K = 100
NBUF = 3
R = 6
BR = 1280


def _mesh():
    return plsc.VectorSubcoreMesh(core_axis_name="c", subcore_axis_name="s")


def _make_deg_kernel(NCH, Npad):
    ZR = Npad // NS
    DNB = 4

    @functools.partial(
        pl.kernel,
        out_type=(
            jax.ShapeDtypeStruct((NC, Npad), jnp.float32),
            jax.ShapeDtypeStruct((NC, Npad), jnp.float32),
        ),
        mesh=_mesh(),
        scratch_types=[
            pltpu.VMEM((NCH, K), jnp.int32),
            pltpu.VMEM((NCH, K), jnp.int32),
            pltpu.VMEM((K,), jnp.float32),
            pltpu.VMEM_SHARED((Npad,), jnp.float32),
            pltpu.VMEM_SHARED((Npad,), jnp.float32),
            pltpu.SemaphoreType.DMA,
        ],
    )
    def deg_kernel(x_hbm, ones_hbm, zz_hbm, od_out, id_out,
                   xs_v, xd_v, ones_v, od_sh, id_sh, sem):
        cid = lax.axis_index("c")
        sid = lax.axis_index("s")
        wid = cid * NS + sid
        pltpu.sync_copy(x_hbm.at[0, wid], xs_v)
        pltpu.sync_copy(x_hbm.at[1, wid], xd_v)
        pltpu.sync_copy(ones_hbm, ones_v)
        pltpu.sync_copy(zz_hbm, od_sh.at[pl.ds(sid * ZR, ZR)])
        pltpu.sync_copy(zz_hbm, id_sh.at[pl.ds(sid * ZR, ZR)])
        plsc.subcore_barrier()

        @pl.loop(0, NCH, step=DNB)
        def _round(i0):
            for b in range(DNB):
                i = i0 + b
                pltpu.async_copy(ones_v, od_sh.at[xs_v.at[i]], sem, add=True)
                pltpu.async_copy(ones_v, id_sh.at[xd_v.at[i]], sem, add=True)
            for b in range(DNB):
                i = i0 + b
                pltpu.make_async_copy(ones_v, od_sh.at[xs_v.at[i]], sem).wait()
                pltpu.make_async_copy(ones_v, id_sh.at[xd_v.at[i]], sem).wait()

        plsc.subcore_barrier()
        pltpu.sync_copy(od_sh.at[pl.ds(sid * ZR, ZR)],
                        od_out.at[cid, pl.ds(sid * ZR, ZR)])
        pltpu.sync_copy(id_sh.at[pl.ds(sid * ZR, ZR)],
                        id_out.at[cid, pl.ds(sid * ZR, ZR)])

    return deg_kernel


def _make_scatter_kernel(NCH, Npad, H):
    ZR = Npad // NS

    @functools.partial(
        pl.kernel,
        out_type=jax.ShapeDtypeStruct((NC, Npad, H), jnp.float32),
        mesh=_mesh(),
        scratch_types=[
            pltpu.VMEM((R, 2, K), jnp.int32),
            pltpu.VMEM((NBUF, K, H), jnp.float32),
            pltpu.VMEM_SHARED((Npad, H), jnp.float32),
            pltpu.SemaphoreType.DMA((NBUF,)),
            pltpu.SemaphoreType.DMA((NBUF,)),
            pltpu.SemaphoreType.DMA((R,)),
        ],
    )
    def scat_kernel(g_hbm, x_hbm, zrows_hbm, out_hbm,
                    x_v, rows_v, acc_sh, gsem, ssem, xsem):
        cid = lax.axis_index("c")
        sid = lax.axis_index("s")
        wid = cid * NS + sid
        pltpu.sync_copy(zrows_hbm, acc_sh.at[pl.ds(sid * ZR, ZR), :])
        plsc.subcore_barrier()

        for c in range(2):
            pltpu.sync_copy(x_hbm.at[0, wid, c], x_v.at[c, 0])
            pltpu.sync_copy(x_hbm.at[1, wid, c], x_v.at[c, 1])
            pltpu.async_copy(g_hbm.at[x_v.at[c, 0]],
                             rows_v.at[c % NBUF], gsem.at[c % NBUF])
        for r in range(2, R):
            pltpu.async_copy(x_hbm.at[0, wid, r], x_v.at[r, 0], xsem.at[r])
            pltpu.async_copy(x_hbm.at[1, wid, r], x_v.at[r, 1], xsem.at[r])

        def _iter(i, q, first, jmax):
            b, r_i = q % NBUF, q % R
            bp, rp = (q - 1) % NBUF, (q - 1) % R
            r_j = (q + 2) % R
            pltpu.make_async_copy(g_hbm.at[x_v.at[r_i, 0]],
                                  rows_v.at[b], gsem.at[b]).wait()
            pltpu.async_copy(rows_v.at[b], acc_sh.at[x_v.at[r_i, 1]],
                             ssem.at[b], add=True)
            if not first:
                pltpu.make_async_copy(rows_v.at[bp],
                                      acc_sh.at[x_v.at[rp, 1]],
                                      ssem.at[bp]).wait()
            j = i + 2
            if jmax is None or j < jmax:
                pltpu.make_async_copy(x_hbm.at[0, wid, j],
                                      x_v.at[r_j, 0], xsem.at[r_j]).wait()
                pltpu.make_async_copy(x_hbm.at[1, wid, j],
                                      x_v.at[r_j, 1], xsem.at[r_j]).wait()
                pltpu.async_copy(g_hbm.at[x_v.at[r_j, 0]],
                                 rows_v.at[bp], gsem.at[bp])
            jj = i + R - 1
            if (jmax is None or jj < jmax) and not first:
                pltpu.async_copy(x_hbm.at[0, wid, jj], x_v.at[rp, 0],
                                 xsem.at[rp])
                pltpu.async_copy(x_hbm.at[1, wid, jj], x_v.at[rp, 1],
                                 xsem.at[rp])

        for i in range(2):
            _iter(i, i, i == 0, NCH)

        ts = 2 + R * ((NCH - R - 1) // R)

        @pl.loop(0, ts - 2, step=R)
        def _round(i0):
            for q in range(R):
                _iter(i0 + q + 2, q + 2, False, None)

        for i in range(ts, NCH):
            _iter(i, i, False, NCH)
        pltpu.make_async_copy(rows_v.at[(NCH - 1) % NBUF],
                              acc_sh.at[x_v.at[(NCH - 1) % R, 1]],
                              ssem.at[(NCH - 1) % NBUF]).wait()

        plsc.subcore_barrier()
        pltpu.sync_copy(acc_sh.at[pl.ds(sid * ZR, ZR), :],
                        out_hbm.at[cid, pl.ds(sid * ZR, ZR), :])

    return scat_kernel


def _ncol(ref):
    return lax.rsqrt(jnp.maximum(ref[0] + ref[1], 1.0))


def _prep_body(f_ref, od_ref, g_ref):
    g_ref[...] = f_ref[...] * _ncol(od_ref)


def _mid_body(p_ref, od_ref, id_ref, w_ref, b_ref, g_ref):
    s = p_ref[0] + p_ref[1]
    h = jnp.dot(s * _ncol(id_ref), w_ref[...],
                preferred_element_type=jnp.float32)
    h = jnp.maximum(h + b_ref[...], 0.0)
    g_ref[...] = h * _ncol(od_ref)


def _fin_body(p_ref, id_ref, w_ref, b_ref, o_ref):
    s = p_ref[0] + p_ref[1]
    h = jnp.dot(s * _ncol(id_ref), w_ref[...],
                preferred_element_type=jnp.float32)
    o_ref[...] = h + b_ref[...]


def _col_spec(br):
    return pl.BlockSpec((NC, br, 1), lambda i: (0, i, 0))


def _row_spec(br, H):
    return pl.BlockSpec((br, H), lambda i: (i, 0))


def _parts_spec(br, H):
    return pl.BlockSpec((NC, br, H), lambda i: (0, i, 0))


def _full_spec(shape):
    nd = len(shape)
    return pl.BlockSpec(shape, lambda i: (0,) * nd)


def kernel(features, edge_index, W0, b0, W1, b1, W2, b2):
    N, F = features.shape
    H = W0.shape[1]
    C = W2.shape[1]
    E = edge_index.shape[1]

    Npad = ((N + BR - 1) // BR) * BR
    NCH = -(-E // (NW * K))
    NCH = max(((NCH + 3) // 4) * 4, 2 * R)
    Epad = NCH * NW * K
    ZR = Npad // NS

    x = edge_index
    if Epad != E:
        fill = N + (jnp.arange(Epad - E, dtype=jnp.int32) % (Npad - N))
        x = jnp.concatenate([x, jnp.stack([fill, fill])], axis=1)
    x4 = x.reshape(2, NW, NCH, K)

    BRN = next(d for d in range(min(BR, N), 7, -1) if N % d == 0 and d % 8 == 0)

    ones_k = jnp.ones((K,), jnp.float32)
    zz1 = jnp.zeros((ZR,), jnp.float32)
    zrows = jnp.zeros((ZR, H), jnp.float32)

    deg_kernel = _make_deg_kernel(NCH, Npad)
    scat_kernel = _make_scatter_kernel(NCH, Npad, H)

    od, idg = deg_kernel(x4, ones_k, zz1)
    od3 = od.reshape(NC, Npad, 1)
    id3 = idg.reshape(NC, Npad, 1)

    g0 = pl.pallas_call(
        _prep_body,
        grid=(N // BRN,),
        in_specs=[_row_spec(BRN, F), _col_spec(BRN)],
        out_specs=_row_spec(BRN, F),
        out_shape=jax.ShapeDtypeStruct((N, F), jnp.float32),
    )(features, od3)

    def mid_layer(g, W, b):
        parts = scat_kernel(g, x4, zrows)
        return pl.pallas_call(
            _mid_body,
            grid=(Npad // BR,),
            in_specs=[_parts_spec(BR, H), _col_spec(BR), _col_spec(BR),
                      _full_spec(W.shape), _full_spec((1, H))],
            out_specs=_row_spec(BR, H),
            out_shape=jax.ShapeDtypeStruct((Npad, H), jnp.float32),
        )(parts, od3, id3, W, b.reshape(1, H))

    g1 = mid_layer(g0, W0, b0)
    g2 = mid_layer(g1, W1, b1)

    parts2 = scat_kernel(g2, x4, zrows)
    return pl.pallas_call(
        _fin_body,
        grid=(N // BRN,),
        in_specs=[_parts_spec(BRN, H), _col_spec(BRN),
                  _full_spec((H, C)), _full_spec((1, C))],
        out_specs=_row_spec(BRN, C),
        out_shape=jax.ShapeDtypeStruct((N, C), jnp.float32),
    )(parts2, id3, W2, b2.reshape(1, C))


# speedup vs baseline: 1.0168x; 1.0168x over previous
import functools

import jax
import jax.numpy as jnp
from jax import lax
from jax.experimental import pallas as pl
from jax.experimental.pallas import tpu as pltpu
from jax.experimental.pallas import tpu_sc as plsc

NC = 2
NS = 16
NW = NC * NS
K = 100
NBUF = 3
R = 6
BR = 2560


def _mesh():
    return plsc.VectorSubcoreMesh(core_axis_name="c", subcore_axis_name="s")


def _make_deg_kernel(NCH, Npad):
    ZR = Npad // NS
    DNB = 4

    @functools.partial(
        pl.kernel,
        out_type=(
            jax.ShapeDtypeStruct((NC, Npad), jnp.float32),
            jax.ShapeDtypeStruct((NC, Npad), jnp.float32),
        ),
        mesh=_mesh(),
        scratch_types=[
            pltpu.VMEM((NCH, K), jnp.int32),
            pltpu.VMEM((NCH, K), jnp.int32),
            pltpu.VMEM((K,), jnp.float32),
            pltpu.VMEM_SHARED((Npad,), jnp.float32),
            pltpu.VMEM_SHARED((Npad,), jnp.float32),
            pltpu.SemaphoreType.DMA,
        ],
    )
    def deg_kernel(x_hbm, ones_hbm, zz_hbm, od_out, id_out,
                   xs_v, xd_v, ones_v, od_sh, id_sh, sem):
        cid = lax.axis_index("c")
        sid = lax.axis_index("s")
        wid = cid * NS + sid
        pltpu.sync_copy(x_hbm.at[0, wid], xs_v)
        pltpu.sync_copy(x_hbm.at[1, wid], xd_v)
        pltpu.sync_copy(ones_hbm, ones_v)
        pltpu.sync_copy(zz_hbm, od_sh.at[pl.ds(sid * ZR, ZR)])
        pltpu.sync_copy(zz_hbm, id_sh.at[pl.ds(sid * ZR, ZR)])
        plsc.subcore_barrier()

        @pl.loop(0, NCH, step=DNB)
        def _round(i0):
            for b in range(DNB):
                i = i0 + b
                pltpu.async_copy(ones_v, od_sh.at[xs_v.at[i]], sem, add=True)
                pltpu.async_copy(ones_v, id_sh.at[xd_v.at[i]], sem, add=True)
            for b in range(DNB):
                i = i0 + b
                pltpu.make_async_copy(ones_v, od_sh.at[xs_v.at[i]], sem).wait()
                pltpu.make_async_copy(ones_v, id_sh.at[xd_v.at[i]], sem).wait()

        plsc.subcore_barrier()
        pltpu.sync_copy(od_sh.at[pl.ds(sid * ZR, ZR)],
                        od_out.at[cid, pl.ds(sid * ZR, ZR)])
        pltpu.sync_copy(id_sh.at[pl.ds(sid * ZR, ZR)],
                        id_out.at[cid, pl.ds(sid * ZR, ZR)])

    return deg_kernel


def _make_scatter_kernel(NCH, Npad, H):
    ZR = Npad // NS

    @functools.partial(
        pl.kernel,
        out_type=jax.ShapeDtypeStruct((NC, Npad, H), jnp.float32),
        mesh=_mesh(),
        scratch_types=[
            pltpu.VMEM((R, 2, K), jnp.int32),
            pltpu.VMEM((NBUF, K, H), jnp.float32),
            pltpu.VMEM_SHARED((Npad, H), jnp.float32),
            pltpu.SemaphoreType.DMA((NBUF,)),
            pltpu.SemaphoreType.DMA((NBUF,)),
            pltpu.SemaphoreType.DMA((R,)),
        ],
    )
    def scat_kernel(g_hbm, x_hbm, zrows_hbm, out_hbm,
                    x_v, rows_v, acc_sh, gsem, ssem, xsem):
        cid = lax.axis_index("c")
        sid = lax.axis_index("s")
        wid = cid * NS + sid
        pltpu.sync_copy(zrows_hbm, acc_sh.at[pl.ds(sid * ZR, ZR), :])
        plsc.subcore_barrier()

        for c in range(2):
            pltpu.sync_copy(x_hbm.at[0, wid, c], x_v.at[c, 0])
            pltpu.sync_copy(x_hbm.at[1, wid, c], x_v.at[c, 1])
            pltpu.async_copy(g_hbm.at[x_v.at[c, 0]],
                             rows_v.at[c % NBUF], gsem.at[c % NBUF])
        for r in range(2, R):
            pltpu.async_copy(x_hbm.at[0, wid, r], x_v.at[r, 0], xsem.at[r])
            pltpu.async_copy(x_hbm.at[1, wid, r], x_v.at[r, 1], xsem.at[r])

        def _iter(i, q, first, jmax):
            b, r_i = q % NBUF, q % R
            bp, rp = (q - 1) % NBUF, (q - 1) % R
            r_j = (q + 2) % R
            pltpu.make_async_copy(g_hbm.at[x_v.at[r_i, 0]],
                                  rows_v.at[b], gsem.at[b]).wait()
            pltpu.async_copy(rows_v.at[b], acc_sh.at[x_v.at[r_i, 1]],
                             ssem.at[b], add=True)
            if not first:
                pltpu.make_async_copy(rows_v.at[bp],
                                      acc_sh.at[x_v.at[rp, 1]],
                                      ssem.at[bp]).wait()
            j = i + 2
            if jmax is None or j < jmax:
                pltpu.make_async_copy(x_hbm.at[0, wid, j],
                                      x_v.at[r_j, 0], xsem.at[r_j]).wait()
                pltpu.make_async_copy(x_hbm.at[1, wid, j],
                                      x_v.at[r_j, 1], xsem.at[r_j]).wait()
                pltpu.async_copy(g_hbm.at[x_v.at[r_j, 0]],
                                 rows_v.at[bp], gsem.at[bp])
            jj = i + R - 1
            if (jmax is None or jj < jmax) and not first:
                pltpu.async_copy(x_hbm.at[0, wid, jj], x_v.at[rp, 0],
                                 xsem.at[rp])
                pltpu.async_copy(x_hbm.at[1, wid, jj], x_v.at[rp, 1],
                                 xsem.at[rp])

        for i in range(2):
            _iter(i, i, i == 0, NCH)

        ts = 2 + R * ((NCH - R - 1) // R)

        @pl.loop(0, ts - 2, step=R)
        def _round(i0):
            for q in range(R):
                _iter(i0 + q + 2, q + 2, False, None)

        for i in range(ts, NCH):
            _iter(i, i, False, NCH)
        pltpu.make_async_copy(rows_v.at[(NCH - 1) % NBUF],
                              acc_sh.at[x_v.at[(NCH - 1) % R, 1]],
                              ssem.at[(NCH - 1) % NBUF]).wait()

        plsc.subcore_barrier()
        pltpu.sync_copy(acc_sh.at[pl.ds(sid * ZR, ZR), :],
                        out_hbm.at[cid, pl.ds(sid * ZR, ZR), :])

    return scat_kernel


def _ncol(ref):
    return lax.rsqrt(jnp.maximum(ref[0] + ref[1], 1.0))


def _prep_body(f_ref, od_ref, g_ref):
    g_ref[...] = f_ref[...] * _ncol(od_ref)


def _mid_body(p_ref, od_ref, id_ref, w_ref, b_ref, g_ref):
    s = p_ref[0] + p_ref[1]
    h = jnp.dot(s * _ncol(id_ref), w_ref[...],
                preferred_element_type=jnp.float32)
    h = jnp.maximum(h + b_ref[...], 0.0)
    g_ref[...] = h * _ncol(od_ref)


def _fin_body(p_ref, id_ref, w_ref, b_ref, o_ref):
    s = p_ref[0] + p_ref[1]
    h = jnp.dot(s * _ncol(id_ref), w_ref[...],
                preferred_element_type=jnp.float32)
    o_ref[...] = h + b_ref[...]


def _col_spec(br):
    return pl.BlockSpec((NC, br, 1), lambda i: (0, i, 0))


def _row_spec(br, H):
    return pl.BlockSpec((br, H), lambda i: (i, 0))


def _parts_spec(br, H):
    return pl.BlockSpec((NC, br, H), lambda i: (0, i, 0))


def _full_spec(shape):
    nd = len(shape)
    return pl.BlockSpec(shape, lambda i: (0,) * nd)


def kernel(features, edge_index, W0, b0, W1, b1, W2, b2):
    N, F = features.shape
    H = W0.shape[1]
    C = W2.shape[1]
    E = edge_index.shape[1]

    Npad = ((N + BR - 1) // BR) * BR
    NCH = -(-E // (NW * K))
    NCH = max(((NCH + 3) // 4) * 4, 2 * R)
    Epad = NCH * NW * K
    ZR = Npad // NS

    x = edge_index
    if Epad != E:
        fill = N + (jnp.arange(Epad - E, dtype=jnp.int32) % (Npad - N))
        x = jnp.concatenate([x, jnp.stack([fill, fill])], axis=1)
    x4 = x.reshape(2, NW, NCH, K)

    BRN = next(d for d in range(min(BR, N), 7, -1) if N % d == 0 and d % 8 == 0)

    ones_k = jnp.ones((K,), jnp.float32)
    zz1 = jnp.zeros((ZR,), jnp.float32)
    zrows = jnp.zeros((ZR, H), jnp.float32)

    deg_kernel = _make_deg_kernel(NCH, Npad)
    scat_kernel = _make_scatter_kernel(NCH, Npad, H)

    od, idg = deg_kernel(x4, ones_k, zz1)
    od3 = od.reshape(NC, Npad, 1)
    id3 = idg.reshape(NC, Npad, 1)

    g0 = pl.pallas_call(
        _prep_body,
        grid=(N // BRN,),
        in_specs=[_row_spec(BRN, F), _col_spec(BRN)],
        out_specs=_row_spec(BRN, F),
        out_shape=jax.ShapeDtypeStruct((N, F), jnp.float32),
    )(features, od3)

    def mid_layer(g, W, b):
        parts = scat_kernel(g, x4, zrows)
        return pl.pallas_call(
            _mid_body,
            grid=(Npad // BR,),
            in_specs=[_parts_spec(BR, H), _col_spec(BR), _col_spec(BR),
                      _full_spec(W.shape), _full_spec((1, H))],
            out_specs=_row_spec(BR, H),
            out_shape=jax.ShapeDtypeStruct((Npad, H), jnp.float32),
        )(parts, od3, id3, W, b.reshape(1, H))

    g1 = mid_layer(g0, W0, b0)
    g2 = mid_layer(g1, W1, b1)

    parts2 = scat_kernel(g2, x4, zrows)
    return pl.pallas_call(
        _fin_body,
        grid=(N // BRN,),
        in_specs=[_parts_spec(BRN, H), _col_spec(BRN),
                  _full_spec((H, C)), _full_spec((1, C))],
        out_specs=_row_spec(BRN, C),
        out_shape=jax.ShapeDtypeStruct((N, C), jnp.float32),
    )(parts2, id3, W2, b2.reshape(1, C))
